# Initial kernel scaffold; baseline (speedup 1.0000x reference)
import jax
import jax.numpy as jnp
from jax.experimental import pallas as pl


def kernel(inDoc, adj, W, a1, a2, clsW, clsb):
    raise NotImplementedError("write your pallas kernel here")


# speedup vs baseline: 2.1326x; 2.1326x over previous
import jax
import jax.numpy as jnp
from jax.experimental import pallas as pl

_N, _IN_FEAT, _S_FEAT, _LABELS, _SLOPE = 4096, 512, 256, 2, 0.01
_BR = 256
_NEG = -9e15


def _proj_kernel(inDoc_ref, W_ref, a1_ref, a2_ref, whb_ref, f1_ref, f2_ref):
    wh = jnp.dot(inDoc_ref[...], W_ref[...], preferred_element_type=jnp.float32)
    whb_ref[...] = wh.astype(jnp.bfloat16)
    f1_ref[...] = jnp.dot(wh, a1_ref[...], preferred_element_type=jnp.float32)
    f2_ref[...] = jnp.dot(wh, a2_ref[...], preferred_element_type=jnp.float32)


def _attn_kernel(f1_ref, f2t_ref, adj_ref, whb_ref, clsW_ref, clsb_ref,
                 att_ref, doc_ref, pool_ref, label_ref):
    i = pl.program_id(0)
    logits = f1_ref[...] + f2t_ref[...]
    logits = jnp.where(logits >= 0, logits, _SLOPE * logits)
    logits = jnp.where(adj_ref[...] > 0, logits, _NEG)
    m = jnp.max(logits, axis=1, keepdims=True)
    p = jnp.exp(logits - m)
    att = p / jnp.sum(p, axis=1, keepdims=True)
    att_ref[...] = att
    doc = jnp.dot(att.astype(jnp.bfloat16), whb_ref[...],
                  preferred_element_type=jnp.float32)
    doc = jnp.where(doc > 0, doc, jnp.exp(doc) - 1.0)
    doc_ref[...] = doc
    colsum = jnp.sum(doc, axis=0, keepdims=True)

    @pl.when(i == 0)
    def _():
        pool_ref[...] = colsum

    @pl.when(i > 0)
    def _():
        pool_ref[...] += colsum

    @pl.when(i == _N // _BR - 1)
    def _():
        pool = pool_ref[...] * (1.0 / _N)
        pool_ref[...] = pool
        cls = jnp.dot(pool, clsW_ref[...],
                      preferred_element_type=jnp.float32) + clsb_ref[...]
        cm = jnp.max(cls, axis=1, keepdims=True)
        cp = jnp.exp(cls - cm)
        label_ref[...] = cp / jnp.sum(cp, axis=1, keepdims=True)


def kernel(inDoc, adj, W, a1, a2, clsW, clsb):
    whb, f1, f2 = pl.pallas_call(
        _proj_kernel,
        out_shape=(
            jax.ShapeDtypeStruct((_N, _S_FEAT), jnp.bfloat16),
            jax.ShapeDtypeStruct((_N, 1), jnp.float32),
            jax.ShapeDtypeStruct((_N, 1), jnp.float32),
        ),
    )(inDoc, W, a1, a2)

    f2t = f2.reshape(1, _N)
    clsb2 = clsb.reshape(1, _LABELS)
    grid = _N // _BR

    att, doc, pool, label = pl.pallas_call(
        _attn_kernel,
        grid=(grid,),
        in_specs=[
            pl.BlockSpec((_BR, 1), lambda i: (i, 0)),
            pl.BlockSpec((1, _N), lambda i: (0, 0)),
            pl.BlockSpec((_BR, _N), lambda i: (i, 0)),
            pl.BlockSpec((_N, _S_FEAT), lambda i: (0, 0)),
            pl.BlockSpec((_S_FEAT, _LABELS), lambda i: (0, 0)),
            pl.BlockSpec((1, _LABELS), lambda i: (0, 0)),
        ],
        out_specs=[
            pl.BlockSpec((_BR, _N), lambda i: (i, 0)),
            pl.BlockSpec((_BR, _S_FEAT), lambda i: (i, 0)),
            pl.BlockSpec((1, _S_FEAT), lambda i: (0, 0)),
            pl.BlockSpec((1, _LABELS), lambda i: (0, 0)),
        ],
        out_shape=(
            jax.ShapeDtypeStruct((_N, _N), jnp.float32),
            jax.ShapeDtypeStruct((_N, _S_FEAT), jnp.float32),
            jax.ShapeDtypeStruct((1, _S_FEAT), jnp.float32),
            jax.ShapeDtypeStruct((1, _LABELS), jnp.float32),
        ),
    )(f1, f2t, adj, whb, clsW, clsb2)

    return (pool.reshape(_S_FEAT), att, doc, label.reshape(_LABELS))


# speedup vs baseline: 2.4832x; 1.1644x over previous
import jax
import jax.numpy as jnp
from jax.experimental import pallas as pl
from jax.experimental.pallas import tpu as pltpu

_N, _IN_FEAT, _S_FEAT, _LABELS, _SLOPE = 4096, 512, 256, 2, 0.01
_BR = 512
_NEG = -9e15


def _gat_kernel(inDoc_ref, W_ref, a1_ref, a2_ref, adj_ref, clsW_ref, clsb_ref,
                att_ref, doc_ref, pool_ref, label_ref,
                whb_ref, f1_ref, f2t_ref):
    i = pl.program_id(0)

    @pl.when(i == 0)
    def _():
        wh = jnp.dot(inDoc_ref[...], W_ref[...],
                     preferred_element_type=jnp.float32)
        whb_ref[...] = wh.astype(jnp.bfloat16)
        f1_ref[...] = jnp.dot(wh, a1_ref[...],
                              preferred_element_type=jnp.float32)
        f2t_ref[...] = jax.lax.dot_general(
            a2_ref[...], wh, (((0,), (1,)), ((), ())),
            preferred_element_type=jnp.float32)

    logits = f1_ref[pl.ds(i * _BR, _BR), :] + f2t_ref[...]
    logits = jnp.maximum(logits, _SLOPE * logits)
    logits = jnp.where(adj_ref[...] > 0, logits, _NEG)
    p = jnp.exp(logits)
    att = p * (1.0 / jnp.sum(p, axis=1, keepdims=True))
    att_ref[...] = att
    doc = jnp.dot(att.astype(jnp.bfloat16), whb_ref[...],
                  preferred_element_type=jnp.float32)
    doc = jnp.where(doc > 0, doc, jnp.exp(doc) - 1.0)
    doc_ref[...] = doc
    colsum = jnp.sum(doc, axis=0, keepdims=True)

    @pl.when(i == 0)
    def _():
        pool_ref[...] = colsum

    @pl.when(i > 0)
    def _():
        pool_ref[...] += colsum

    @pl.when(i == _N // _BR - 1)
    def _():
        pool = pool_ref[...] * (1.0 / _N)
        pool_ref[...] = pool
        cls = jnp.dot(pool, clsW_ref[...],
                      preferred_element_type=jnp.float32) + clsb_ref[...]
        cm = jnp.max(cls, axis=1, keepdims=True)
        cp = jnp.exp(cls - cm)
        label_ref[...] = cp / jnp.sum(cp, axis=1, keepdims=True)


def kernel(inDoc, adj, W, a1, a2, clsW, clsb):
    clsb2 = clsb.reshape(1, _LABELS)
    grid = _N // _BR

    att, doc, pool, label = pl.pallas_call(
        _gat_kernel,
        grid=(grid,),
        in_specs=[
            pl.BlockSpec((_N, _IN_FEAT), lambda i: (0, 0)),
            pl.BlockSpec((_IN_FEAT, _S_FEAT), lambda i: (0, 0)),
            pl.BlockSpec((_S_FEAT, 1), lambda i: (0, 0)),
            pl.BlockSpec((_S_FEAT, 1), lambda i: (0, 0)),
            pl.BlockSpec((_BR, _N), lambda i: (i, 0)),
            pl.BlockSpec((_S_FEAT, _LABELS), lambda i: (0, 0)),
            pl.BlockSpec((1, _LABELS), lambda i: (0, 0)),
        ],
        out_specs=[
            pl.BlockSpec((_BR, _N), lambda i: (i, 0)),
            pl.BlockSpec((_BR, _S_FEAT), lambda i: (i, 0)),
            pl.BlockSpec((1, _S_FEAT), lambda i: (0, 0)),
            pl.BlockSpec((1, _LABELS), lambda i: (0, 0)),
        ],
        out_shape=(
            jax.ShapeDtypeStruct((_N, _N), jnp.float32),
            jax.ShapeDtypeStruct((_N, _S_FEAT), jnp.float32),
            jax.ShapeDtypeStruct((1, _S_FEAT), jnp.float32),
            jax.ShapeDtypeStruct((1, _LABELS), jnp.float32),
        ),
        scratch_shapes=[
            pltpu.VMEM((_N, _S_FEAT), jnp.bfloat16),
            pltpu.VMEM((_N, 1), jnp.float32),
            pltpu.VMEM((1, _N), jnp.float32),
        ],
    )(inDoc, W, a1, a2, adj, clsW, clsb2)

    return (pool.reshape(_S_FEAT), att, doc, label.reshape(_LABELS))
